# Initial kernel scaffold; baseline (speedup 1.0000x reference)
import jax
import jax.numpy as jnp
from jax.experimental import pallas as pl


def kernel(x, edge_index, batch, h0, c0, W1a, b1a, W2a, b2a, W1b, b1b, W2b, b2b, W_ih, W_hh, b_ih, b_hh, Wf, bf):
    raise NotImplementedError("write your pallas kernel here")


# speedup vs baseline: 6.9546x; 6.9546x over previous
import functools

import jax
import jax.numpy as jnp
from jax import lax
from jax.experimental import pallas as pl
from jax.experimental.pallas import tpu as pltpu
from jax.experimental.pallas import tpu_sc as plsc

N = 10000
E = 320000
HD = 128
B = 16
LL = 128
A = 64

NC = 2
NS = 16
NW = NC * NS
GSZ = 128
NGW = 80
EPAD = NW * NGW * GSZ
NG = EPAD // GSZ
NROWS = 10112
ZROWS = NROWS // NS
OROWS = 624
TAIL = N - NS * OROWS

_HIGH = jax.lax.Precision.HIGHEST


def _dot(a, b, dims):
    return lax.dot_general(a, b, (dims, ((), ())), precision=_HIGH,
                           preferred_element_type=jnp.float32)


def _sc_agg_body(x_hbm, src_hbm, dst_hbm, zeros_hbm, p0_hbm, p1_hbm,
                 agg, srcv, dstv, rows, sem):
    c = lax.axis_index("c")
    s = lax.axis_index("s")
    w = c * NS + s

    pltpu.sync_copy(zeros_hbm, agg.at[pl.ds(s * ZROWS, ZROWS)])

    pltpu.sync_copy(src_hbm.at[pl.ds(w * NGW, NGW)], srcv)
    pltpu.sync_copy(dst_hbm.at[pl.ds(w * NGW, NGW)], dstv)
    plsc.subcore_barrier()

    def group(g, carry):
        pltpu.async_copy(x_hbm.at[srcv.at[g]], rows, sem).wait()
        pltpu.sync_copy(rows, agg.at[dstv.at[g]], add=True)
        return carry

    lax.fori_loop(0, NGW, group, 0)
    plsc.subcore_barrier()

    @pl.when(c == 0)
    def _():
        pltpu.sync_copy(agg.at[pl.ds(s * OROWS, OROWS)],
                        p0_hbm.at[pl.ds(s * OROWS, OROWS)])

        @pl.when(s == 0)
        def _():
            pltpu.sync_copy(agg.at[pl.ds(NS * OROWS, TAIL)],
                            p0_hbm.at[pl.ds(NS * OROWS, TAIL)])

    @pl.when(c == 1)
    def _():
        pltpu.sync_copy(agg.at[pl.ds(s * OROWS, OROWS)],
                        p1_hbm.at[pl.ds(s * OROWS, OROWS)])

        @pl.when(s == 0)
        def _():
            pltpu.sync_copy(agg.at[pl.ds(NS * OROWS, TAIL)],
                            p1_hbm.at[pl.ds(NS * OROWS, TAIL)])


_sc_agg = pl.kernel(
    _sc_agg_body,
    out_type=(jax.ShapeDtypeStruct((N, HD), jnp.float32),
              jax.ShapeDtypeStruct((N, HD), jnp.float32)),
    mesh=plsc.VectorSubcoreMesh(core_axis_name="c", subcore_axis_name="s",
                                num_cores=NC, num_subcores=NS),
    scratch_types=[
        pltpu.VMEM_SHARED((NROWS, HD), jnp.float32),
        pltpu.VMEM((NGW, GSZ), jnp.int32),
        pltpu.VMEM((NGW, GSZ), jnp.int32),
        pltpu.VMEM((GSZ, HD), jnp.float32),
        pltpu.SemaphoreType.DMA,
    ],
)


def _mlp_block(h, w1_ref, b1_ref, w2_ref, b2_ref):
    t = jnp.maximum(_dot(h, w1_ref[...], (((1,), (0,)))) + b1_ref[...], 0.0)
    return jnp.maximum(_dot(t, w2_ref[...], (((1,), (0,)))) + b2_ref[...], 0.0)


def _tc_mlp_body(x_ref, p0_ref, p1_ref, w1_ref, b1_ref, w2_ref, b2_ref, o_ref):
    h = x_ref[...] + p0_ref[...] + p1_ref[...]
    o_ref[...] = _mlp_block(h, w1_ref, b1_ref, w2_ref, b2_ref)


def _tc_head_body(h_ref, q0_ref, q1_ref, w1_ref, b1_ref, w2_ref, b2_ref,
                  batch_ref, hp_ref, cp_ref, wih_ref, whh_ref, bih_ref,
                  wf_ref, bf_ref, probs_ref, hn_ref, cn_ref,
                  pooled_acc, cnt_acc, BN):
    i = pl.program_id(0)

    @pl.when(i == 0)
    def _():
        pooled_acc[...] = jnp.zeros_like(pooled_acc)
        cnt_acc[...] = jnp.zeros_like(cnt_acc)

    h = h_ref[...] + q0_ref[...] + q1_ref[...]
    h2 = _mlp_block(h, w1_ref, b1_ref, w2_ref, b2_ref)

    mask = (batch_ref[0] == lax.broadcasted_iota(jnp.int32, (B, BN), 0))
    mask = mask.astype(jnp.float32)
    pooled_acc[...] += _dot(mask, h2, (((1,), (0,))))
    cnt_acc[...] += _dot(mask, jnp.ones((BN, HD), jnp.float32), (((1,), (0,))))

    @pl.when(i == pl.num_programs(0) - 1)
    def _():
        pooled = pooled_acc[...] / jnp.maximum(cnt_acc[...], 1.0)
        gates = (_dot(pooled, wih_ref[...], (((1,), (0,))))
                 + _dot(hp_ref[...], whh_ref[...], (((1,), (0,))))
                 + bih_ref[...])
        i_g = jax.nn.sigmoid(gates[:, 0 * LL:1 * LL])
        f_g = jax.nn.sigmoid(gates[:, 1 * LL:2 * LL])
        g_g = jnp.tanh(gates[:, 2 * LL:3 * LL])
        o_g = jax.nn.sigmoid(gates[:, 3 * LL:4 * LL])
        c_new = f_g * cp_ref[...] + i_g * g_g
        h_new = o_g * jnp.tanh(c_new)
        logits = _dot(h_new, wf_ref[...], (((1, ), (0,)))) + bf_ref[...]
        m = jnp.max(logits, axis=1, keepdims=True)
        e = jnp.exp(logits - m)
        probs_ref[...] = e / jnp.sum(e, axis=1, keepdims=True)
        hn_ref[...] = h_new
        cn_ref[...] = c_new


def _tc_mlp(x, p0, p1, w1, b1, w2, b2, nblk, bn):
    row = lambda i: (i, 0)
    full = lambda i: (0, 0)
    return pl.pallas_call(
        _tc_mlp_body,
        grid=(nblk,),
        in_specs=[pl.BlockSpec((bn, HD), row)] * 3 + [
            pl.BlockSpec((HD, HD), full), pl.BlockSpec((1, HD), full),
            pl.BlockSpec((HD, HD), full), pl.BlockSpec((1, HD), full),
        ],
        out_specs=pl.BlockSpec((bn, HD), row),
        out_shape=jax.ShapeDtypeStruct((N, HD), jnp.float32),
    )(x, p0, p1, w1, b1, w2, b2)


def _tc_head(h1, q0, q1, w1, b1, w2, b2, batch3d, hp, cp, wihT, whhT, bihs,
             wf, bf, nblk, bn):
    row = lambda i: (i, 0)
    full = lambda i: (0, 0)
    return pl.pallas_call(
        functools.partial(_tc_head_body, BN=bn),
        grid=(nblk,),
        in_specs=[pl.BlockSpec((bn, HD), row)] * 3 + [
            pl.BlockSpec((HD, HD), full), pl.BlockSpec((1, HD), full),
            pl.BlockSpec((HD, HD), full), pl.BlockSpec((1, HD), full),
            pl.BlockSpec((1, 1, bn), lambda i: (i, 0, 0)),
            pl.BlockSpec((B, LL), full), pl.BlockSpec((B, LL), full),
            pl.BlockSpec((HD, 4 * LL), full), pl.BlockSpec((LL, 4 * LL), full),
            pl.BlockSpec((1, 4 * LL), full),
            pl.BlockSpec((LL, A), full), pl.BlockSpec((1, A), full),
        ],
        out_specs=[pl.BlockSpec((B, A), full), pl.BlockSpec((B, LL), full),
                   pl.BlockSpec((B, LL), full)],
        out_shape=[jax.ShapeDtypeStruct((B, A), jnp.float32),
                   jax.ShapeDtypeStruct((B, LL), jnp.float32),
                   jax.ShapeDtypeStruct((B, LL), jnp.float32)],
        scratch_shapes=[pltpu.VMEM((B, HD), jnp.float32),
                        pltpu.VMEM((B, HD), jnp.float32)],
    )(h1, q0, q1, w1, b1, w2, b2, batch3d, hp, cp, wihT, whhT, bihs, wf, bf)


def kernel(x, edge_index, batch, h0, c0,
           W1a, b1a, W2a, b2a, W1b, b1b, W2b, b2b,
           W_ih, W_hh, b_ih, b_hh, Wf, bf):
    pad = EPAD - E
    padi = jnp.arange(pad, dtype=jnp.int32)
    src_p = jnp.concatenate([edge_index[0], (padi * 131) % N])
    dst_p = jnp.concatenate([edge_index[1], N + (padi % (NROWS - N))])
    src2d = src_p.reshape(NG, GSZ)
    dst2d = dst_p.reshape(NG, GSZ)
    zeros = jnp.zeros((ZROWS, HD), jnp.float32)

    nblk, bn = 10, 1000
    batch3d = batch.reshape(nblk, 1, bn)
    b1as, b2as = b1a.reshape(1, HD), b2a.reshape(1, HD)
    b1bs, b2bs = b1b.reshape(1, HD), b2b.reshape(1, HD)
    bihs = (b_ih + b_hh).reshape(1, 4 * LL)
    bfs = bf.reshape(1, A)

    p0, p1 = _sc_agg(x, src2d, dst2d, zeros)
    h1 = _tc_mlp(x, p0, p1, W1a, b1as, W2a, b2as, nblk, bn)

    q0, q1 = _sc_agg(h1, src2d, dst2d, zeros)
    probs, h_new, c_new = _tc_head(h1, q0, q1, W1b, b1bs, W2b, b2bs,
                                   batch3d, h0[0], c0[0], W_ih.T, W_hh.T,
                                   bihs, Wf, bfs, nblk, bn)
    return probs, h_new[None], c_new[None]


# speedup vs baseline: 9.3114x; 1.3389x over previous
import functools

import jax
import jax.numpy as jnp
from jax import lax
from jax.experimental import pallas as pl
from jax.experimental.pallas import tpu as pltpu
from jax.experimental.pallas import tpu_sc as plsc

N = 10000
E = 320000
HD = 128
B = 16
LL = 128
A = 64

NC = 2
NS = 16
NW = NC * NS
GSZ = 128
NGW = 80
EPAD = NW * NGW * GSZ
NG = EPAD // GSZ
NROWS = 10112
ZROWS = NROWS // NS
OROWS = 624
TAIL = N - NS * OROWS

_HIGH = jax.lax.Precision.HIGHEST


def _dot(a, b, dims):
    return lax.dot_general(a, b, (dims, ((), ())), precision=_HIGH,
                           preferred_element_type=jnp.float32)


NBUF = 2
IB = 16
NCHUNK = NGW // IB


def _sc_agg_body(x_hbm, src_hbm, dst_hbm, zeros_hbm, p0_hbm, p1_hbm,
                 agg, sbuf, dbuf, rows, sems):
    c = lax.axis_index("c")
    s = lax.axis_index("s")
    w = c * NS + s

    pltpu.sync_copy(zeros_hbm, agg.at[pl.ds(s * ZROWS, ZROWS)])
    plsc.subcore_barrier()

    for ci in range(NCHUNK):
        cb = w * NGW + ci * IB
        pltpu.sync_copy(src_hbm.at[pl.ds(cb, IB)], sbuf)
        pltpu.sync_copy(dst_hbm.at[pl.ds(cb, IB)], dbuf)
        for b in range(NBUF):
            pltpu.make_async_copy(x_hbm.at[sbuf.at[b]], rows[b], sems[b]).start()

        def pair(j, carry):
            for b in range(NBUF):
                g = NBUF * j + b
                pltpu.make_async_copy(x_hbm.at[sbuf.at[g]], rows[b],
                                      sems[b]).wait()
                pltpu.sync_copy(rows[b], agg.at[dbuf.at[g]], add=True)

                @pl.when(g + NBUF < IB)
                def _(g=g, b=b):
                    pltpu.make_async_copy(x_hbm.at[sbuf.at[g + NBUF]], rows[b],
                                          sems[b]).start()
            return carry

        lax.fori_loop(0, IB // NBUF, pair, 0)

    plsc.subcore_barrier()

    @pl.when(c == 0)
    def _():
        pltpu.sync_copy(agg.at[pl.ds(s * OROWS, OROWS)],
                        p0_hbm.at[pl.ds(s * OROWS, OROWS)])

        @pl.when(s == 0)
        def _():
            pltpu.sync_copy(agg.at[pl.ds(NS * OROWS, TAIL)],
                            p0_hbm.at[pl.ds(NS * OROWS, TAIL)])

    @pl.when(c == 1)
    def _():
        pltpu.sync_copy(agg.at[pl.ds(s * OROWS, OROWS)],
                        p1_hbm.at[pl.ds(s * OROWS, OROWS)])

        @pl.when(s == 0)
        def _():
            pltpu.sync_copy(agg.at[pl.ds(NS * OROWS, TAIL)],
                            p1_hbm.at[pl.ds(NS * OROWS, TAIL)])


_sc_agg = pl.kernel(
    _sc_agg_body,
    out_type=(jax.ShapeDtypeStruct((N, HD), jnp.float32),
              jax.ShapeDtypeStruct((N, HD), jnp.float32)),
    mesh=plsc.VectorSubcoreMesh(core_axis_name="c", subcore_axis_name="s",
                                num_cores=NC, num_subcores=NS),
    scratch_types=[
        pltpu.VMEM_SHARED((NROWS, HD), jnp.float32),
        pltpu.VMEM((IB, GSZ), jnp.int32),
        pltpu.VMEM((IB, GSZ), jnp.int32),
        [pltpu.VMEM((GSZ, HD), jnp.float32)] * NBUF,
        [pltpu.SemaphoreType.DMA] * NBUF,
    ],
)


def _mlp_block(h, w1_ref, b1_ref, w2_ref, b2_ref):
    t = jnp.maximum(_dot(h, w1_ref[...], (((1,), (0,)))) + b1_ref[...], 0.0)
    return jnp.maximum(_dot(t, w2_ref[...], (((1,), (0,)))) + b2_ref[...], 0.0)


def _tc_mlp_body(x_ref, p0_ref, p1_ref, w1_ref, b1_ref, w2_ref, b2_ref, o_ref):
    h = x_ref[...] + p0_ref[...] + p1_ref[...]
    o_ref[...] = _mlp_block(h, w1_ref, b1_ref, w2_ref, b2_ref)


def _tc_head_body(h_ref, q0_ref, q1_ref, w1_ref, b1_ref, w2_ref, b2_ref,
                  batch_ref, hp_ref, cp_ref, wih_ref, whh_ref, bih_ref,
                  wf_ref, bf_ref, probs_ref, hn_ref, cn_ref,
                  pooled_acc, cnt_acc, BN):
    i = pl.program_id(0)

    @pl.when(i == 0)
    def _():
        pooled_acc[...] = jnp.zeros_like(pooled_acc)
        cnt_acc[...] = jnp.zeros_like(cnt_acc)

    h = h_ref[...] + q0_ref[...] + q1_ref[...]
    h2 = _mlp_block(h, w1_ref, b1_ref, w2_ref, b2_ref)

    mask = (batch_ref[0] == lax.broadcasted_iota(jnp.int32, (B, BN), 0))
    mask = mask.astype(jnp.float32)
    pooled_acc[...] += _dot(mask, h2, (((1,), (0,))))
    cnt_acc[...] += _dot(mask, jnp.ones((BN, HD), jnp.float32), (((1,), (0,))))

    @pl.when(i == pl.num_programs(0) - 1)
    def _():
        pooled = pooled_acc[...] / jnp.maximum(cnt_acc[...], 1.0)
        gates = (_dot(pooled, wih_ref[...], (((1,), (0,))))
                 + _dot(hp_ref[...], whh_ref[...], (((1,), (0,))))
                 + bih_ref[...])
        i_g = jax.nn.sigmoid(gates[:, 0 * LL:1 * LL])
        f_g = jax.nn.sigmoid(gates[:, 1 * LL:2 * LL])
        g_g = jnp.tanh(gates[:, 2 * LL:3 * LL])
        o_g = jax.nn.sigmoid(gates[:, 3 * LL:4 * LL])
        c_new = f_g * cp_ref[...] + i_g * g_g
        h_new = o_g * jnp.tanh(c_new)
        logits = _dot(h_new, wf_ref[...], (((1, ), (0,)))) + bf_ref[...]
        m = jnp.max(logits, axis=1, keepdims=True)
        e = jnp.exp(logits - m)
        probs_ref[...] = e / jnp.sum(e, axis=1, keepdims=True)
        hn_ref[...] = h_new
        cn_ref[...] = c_new


def _tc_mlp(x, p0, p1, w1, b1, w2, b2, nblk, bn):
    row = lambda i: (i, 0)
    full = lambda i: (0, 0)
    return pl.pallas_call(
        _tc_mlp_body,
        grid=(nblk,),
        in_specs=[pl.BlockSpec((bn, HD), row)] * 3 + [
            pl.BlockSpec((HD, HD), full), pl.BlockSpec((1, HD), full),
            pl.BlockSpec((HD, HD), full), pl.BlockSpec((1, HD), full),
        ],
        out_specs=pl.BlockSpec((bn, HD), row),
        out_shape=jax.ShapeDtypeStruct((N, HD), jnp.float32),
    )(x, p0, p1, w1, b1, w2, b2)


def _tc_head(h1, q0, q1, w1, b1, w2, b2, batch3d, hp, cp, wihT, whhT, bihs,
             wf, bf, nblk, bn):
    row = lambda i: (i, 0)
    full = lambda i: (0, 0)
    return pl.pallas_call(
        functools.partial(_tc_head_body, BN=bn),
        grid=(nblk,),
        in_specs=[pl.BlockSpec((bn, HD), row)] * 3 + [
            pl.BlockSpec((HD, HD), full), pl.BlockSpec((1, HD), full),
            pl.BlockSpec((HD, HD), full), pl.BlockSpec((1, HD), full),
            pl.BlockSpec((1, 1, bn), lambda i: (i, 0, 0)),
            pl.BlockSpec((B, LL), full), pl.BlockSpec((B, LL), full),
            pl.BlockSpec((HD, 4 * LL), full), pl.BlockSpec((LL, 4 * LL), full),
            pl.BlockSpec((1, 4 * LL), full),
            pl.BlockSpec((LL, A), full), pl.BlockSpec((1, A), full),
        ],
        out_specs=[pl.BlockSpec((B, A), full), pl.BlockSpec((B, LL), full),
                   pl.BlockSpec((B, LL), full)],
        out_shape=[jax.ShapeDtypeStruct((B, A), jnp.float32),
                   jax.ShapeDtypeStruct((B, LL), jnp.float32),
                   jax.ShapeDtypeStruct((B, LL), jnp.float32)],
        scratch_shapes=[pltpu.VMEM((B, HD), jnp.float32),
                        pltpu.VMEM((B, HD), jnp.float32)],
    )(h1, q0, q1, w1, b1, w2, b2, batch3d, hp, cp, wihT, whhT, bihs, wf, bf)


def kernel(x, edge_index, batch, h0, c0,
           W1a, b1a, W2a, b2a, W1b, b1b, W2b, b2b,
           W_ih, W_hh, b_ih, b_hh, Wf, bf):
    pad = EPAD - E
    padi = jnp.arange(pad, dtype=jnp.int32)
    src_p = jnp.concatenate([edge_index[0], (padi * 131) % N])
    dst_p = jnp.concatenate([edge_index[1], N + (padi % (NROWS - N))])
    src2d = src_p.reshape(NG, GSZ)
    dst2d = dst_p.reshape(NG, GSZ)
    zeros = jnp.zeros((ZROWS, HD), jnp.float32)

    nblk, bn = 10, 1000
    batch3d = batch.reshape(nblk, 1, bn)
    b1as, b2as = b1a.reshape(1, HD), b2a.reshape(1, HD)
    b1bs, b2bs = b1b.reshape(1, HD), b2b.reshape(1, HD)
    bihs = (b_ih + b_hh).reshape(1, 4 * LL)
    bfs = bf.reshape(1, A)

    p0, p1 = _sc_agg(x, src2d, dst2d, zeros)
    h1 = _tc_mlp(x, p0, p1, W1a, b1as, W2a, b2as, nblk, bn)

    q0, q1 = _sc_agg(h1, src2d, dst2d, zeros)
    probs, h_new, c_new = _tc_head(h1, q0, q1, W1b, b1bs, W2b, b2bs,
                                   batch3d, h0[0], c0[0], W_ih.T, W_hh.T,
                                   bihs, Wf, bfs, nblk, bn)
    return probs, h_new[None], c_new[None]


# speedup vs baseline: 9.8046x; 1.0530x over previous
import functools

import jax
import jax.numpy as jnp
from jax import lax
from jax.experimental import pallas as pl
from jax.experimental.pallas import tpu as pltpu
from jax.experimental.pallas import tpu_sc as plsc

N = 10000
E = 320000
HD = 128
B = 16
LL = 128
A = 64

NC = 2
NS = 16
NW = NC * NS
GSZ = 128
NGW = 80
EPAD = NW * NGW * GSZ
NG = EPAD // GSZ
NROWS = 10112
ZROWS = NROWS // NS
OROWS = 624
TAIL = N - NS * OROWS

_HIGH = jax.lax.Precision.HIGHEST


def _dot(a, b, dims):
    return lax.dot_general(a, b, (dims, ((), ())), precision=_HIGH,
                           preferred_element_type=jnp.float32)


NBUF = 2
IB = 40
NCHUNK = NGW // IB


def _sc_agg_body(x_hbm, src_hbm, dst_hbm, zeros_hbm, p0_hbm, p1_hbm,
                 agg, sbuf, dbuf, rows, sems):
    c = lax.axis_index("c")
    s = lax.axis_index("s")
    w = c * NS + s

    pltpu.sync_copy(zeros_hbm, agg.at[pl.ds(s * ZROWS, ZROWS)])
    plsc.subcore_barrier()

    for ci in range(NCHUNK):
        cb = w * NGW + ci * IB
        pltpu.sync_copy(src_hbm.at[pl.ds(cb, IB)], sbuf)
        pltpu.sync_copy(dst_hbm.at[pl.ds(cb, IB)], dbuf)
        for b in range(NBUF):
            pltpu.make_async_copy(x_hbm.at[sbuf.at[b]], rows[b], sems[b]).start()

        def pair(j, carry):
            for b in range(NBUF):
                g = NBUF * j + b
                pltpu.make_async_copy(x_hbm.at[sbuf.at[g]], rows[b],
                                      sems[b]).wait()
                pltpu.sync_copy(rows[b], agg.at[dbuf.at[g]], add=True)

                @pl.when(g + NBUF < IB)
                def _(g=g, b=b):
                    pltpu.make_async_copy(x_hbm.at[sbuf.at[g + NBUF]], rows[b],
                                          sems[b]).start()
            return carry

        lax.fori_loop(0, IB // NBUF, pair, 0)

    plsc.subcore_barrier()

    @pl.when(c == 0)
    def _():
        pltpu.sync_copy(agg.at[pl.ds(s * OROWS, OROWS)],
                        p0_hbm.at[pl.ds(s * OROWS, OROWS)])

        @pl.when(s == 0)
        def _():
            pltpu.sync_copy(agg.at[pl.ds(NS * OROWS, TAIL)],
                            p0_hbm.at[pl.ds(NS * OROWS, TAIL)])

    @pl.when(c == 1)
    def _():
        pltpu.sync_copy(agg.at[pl.ds(s * OROWS, OROWS)],
                        p1_hbm.at[pl.ds(s * OROWS, OROWS)])

        @pl.when(s == 0)
        def _():
            pltpu.sync_copy(agg.at[pl.ds(NS * OROWS, TAIL)],
                            p1_hbm.at[pl.ds(NS * OROWS, TAIL)])


_sc_agg = pl.kernel(
    _sc_agg_body,
    out_type=(jax.ShapeDtypeStruct((N, HD), jnp.float32),
              jax.ShapeDtypeStruct((N, HD), jnp.float32)),
    mesh=plsc.VectorSubcoreMesh(core_axis_name="c", subcore_axis_name="s",
                                num_cores=NC, num_subcores=NS),
    scratch_types=[
        pltpu.VMEM_SHARED((NROWS, HD), jnp.float32),
        pltpu.VMEM((IB, GSZ), jnp.int32),
        pltpu.VMEM((IB, GSZ), jnp.int32),
        [pltpu.VMEM((GSZ, HD), jnp.float32)] * NBUF,
        [pltpu.SemaphoreType.DMA] * NBUF,
    ],
)


def _mlp_block(h, w1_ref, b1_ref, w2_ref, b2_ref):
    t = jnp.maximum(_dot(h, w1_ref[...], (((1,), (0,)))) + b1_ref[...], 0.0)
    return jnp.maximum(_dot(t, w2_ref[...], (((1,), (0,)))) + b2_ref[...], 0.0)


def _tc_mlp_body(x_ref, p0_ref, p1_ref, w1_ref, b1_ref, w2_ref, b2_ref, o_ref):
    h = x_ref[...] + p0_ref[...] + p1_ref[...]
    o_ref[...] = _mlp_block(h, w1_ref, b1_ref, w2_ref, b2_ref)


def _tc_head_body(h_ref, q0_ref, q1_ref, w1_ref, b1_ref, w2_ref, b2_ref,
                  batch_ref, hp_ref, cp_ref, wih_ref, whh_ref, bih_ref,
                  wf_ref, bf_ref, probs_ref, hn_ref, cn_ref,
                  pooled_acc, cnt_acc, BN):
    i = pl.program_id(0)

    @pl.when(i == 0)
    def _():
        pooled_acc[...] = jnp.zeros_like(pooled_acc)
        cnt_acc[...] = jnp.zeros_like(cnt_acc)

    h = h_ref[...] + q0_ref[...] + q1_ref[...]
    h2 = _mlp_block(h, w1_ref, b1_ref, w2_ref, b2_ref)

    mask = (batch_ref[0] == lax.broadcasted_iota(jnp.int32, (B, BN), 0))
    mask = mask.astype(jnp.float32)
    pooled_acc[...] += _dot(mask, h2, (((1,), (0,))))
    cnt_acc[...] += _dot(mask, jnp.ones((BN, HD), jnp.float32), (((1,), (0,))))

    @pl.when(i == pl.num_programs(0) - 1)
    def _():
        pooled = pooled_acc[...] / jnp.maximum(cnt_acc[...], 1.0)
        gates = (_dot(pooled, wih_ref[...], (((1,), (0,))))
                 + _dot(hp_ref[...], whh_ref[...], (((1,), (0,))))
                 + bih_ref[...])
        i_g = jax.nn.sigmoid(gates[:, 0 * LL:1 * LL])
        f_g = jax.nn.sigmoid(gates[:, 1 * LL:2 * LL])
        g_g = jnp.tanh(gates[:, 2 * LL:3 * LL])
        o_g = jax.nn.sigmoid(gates[:, 3 * LL:4 * LL])
        c_new = f_g * cp_ref[...] + i_g * g_g
        h_new = o_g * jnp.tanh(c_new)
        logits = _dot(h_new, wf_ref[...], (((1, ), (0,)))) + bf_ref[...]
        m = jnp.max(logits, axis=1, keepdims=True)
        e = jnp.exp(logits - m)
        probs_ref[...] = e / jnp.sum(e, axis=1, keepdims=True)
        hn_ref[...] = h_new
        cn_ref[...] = c_new


def _tc_mlp(x, p0, p1, w1, b1, w2, b2, nblk, bn):
    row = lambda i: (i, 0)
    full = lambda i: (0, 0)
    return pl.pallas_call(
        _tc_mlp_body,
        grid=(nblk,),
        in_specs=[pl.BlockSpec((bn, HD), row)] * 3 + [
            pl.BlockSpec((HD, HD), full), pl.BlockSpec((1, HD), full),
            pl.BlockSpec((HD, HD), full), pl.BlockSpec((1, HD), full),
        ],
        out_specs=pl.BlockSpec((bn, HD), row),
        out_shape=jax.ShapeDtypeStruct((N, HD), jnp.float32),
    )(x, p0, p1, w1, b1, w2, b2)


def _tc_head(h1, q0, q1, w1, b1, w2, b2, batch3d, hp, cp, wihT, whhT, bihs,
             wf, bf, nblk, bn):
    row = lambda i: (i, 0)
    full = lambda i: (0, 0)
    return pl.pallas_call(
        functools.partial(_tc_head_body, BN=bn),
        grid=(nblk,),
        in_specs=[pl.BlockSpec((bn, HD), row)] * 3 + [
            pl.BlockSpec((HD, HD), full), pl.BlockSpec((1, HD), full),
            pl.BlockSpec((HD, HD), full), pl.BlockSpec((1, HD), full),
            pl.BlockSpec((1, 1, bn), lambda i: (i, 0, 0)),
            pl.BlockSpec((B, LL), full), pl.BlockSpec((B, LL), full),
            pl.BlockSpec((HD, 4 * LL), full), pl.BlockSpec((LL, 4 * LL), full),
            pl.BlockSpec((1, 4 * LL), full),
            pl.BlockSpec((LL, A), full), pl.BlockSpec((1, A), full),
        ],
        out_specs=[pl.BlockSpec((B, A), full), pl.BlockSpec((B, LL), full),
                   pl.BlockSpec((B, LL), full)],
        out_shape=[jax.ShapeDtypeStruct((B, A), jnp.float32),
                   jax.ShapeDtypeStruct((B, LL), jnp.float32),
                   jax.ShapeDtypeStruct((B, LL), jnp.float32)],
        scratch_shapes=[pltpu.VMEM((B, HD), jnp.float32),
                        pltpu.VMEM((B, HD), jnp.float32)],
    )(h1, q0, q1, w1, b1, w2, b2, batch3d, hp, cp, wihT, whhT, bihs, wf, bf)


def kernel(x, edge_index, batch, h0, c0,
           W1a, b1a, W2a, b2a, W1b, b1b, W2b, b2b,
           W_ih, W_hh, b_ih, b_hh, Wf, bf):
    pad = EPAD - E
    padi = jnp.arange(pad, dtype=jnp.int32)
    src_p = jnp.concatenate([edge_index[0], (padi * 131) % N])
    dst_p = jnp.concatenate([edge_index[1], N + (padi % (NROWS - N))])
    src2d = src_p.reshape(NG, GSZ)
    dst2d = dst_p.reshape(NG, GSZ)
    zeros = jnp.zeros((ZROWS, HD), jnp.float32)

    nblk, bn = 10, 1000
    batch3d = batch.reshape(nblk, 1, bn)
    b1as, b2as = b1a.reshape(1, HD), b2a.reshape(1, HD)
    b1bs, b2bs = b1b.reshape(1, HD), b2b.reshape(1, HD)
    bihs = (b_ih + b_hh).reshape(1, 4 * LL)
    bfs = bf.reshape(1, A)

    p0, p1 = _sc_agg(x, src2d, dst2d, zeros)
    h1 = _tc_mlp(x, p0, p1, W1a, b1as, W2a, b2as, nblk, bn)

    q0, q1 = _sc_agg(h1, src2d, dst2d, zeros)
    probs, h_new, c_new = _tc_head(h1, q0, q1, W1b, b1bs, W2b, b2bs,
                                   batch3d, h0[0], c0[0], W_ih.T, W_hh.T,
                                   bihs, Wf, bfs, nblk, bn)
    return probs, h_new[None], c_new[None]


# speedup vs baseline: 10.0869x; 1.0288x over previous
import functools

import jax
import jax.numpy as jnp
from jax import lax
from jax.experimental import pallas as pl
from jax.experimental.pallas import tpu as pltpu
from jax.experimental.pallas import tpu_sc as plsc

N = 10000
E = 320000
HD = 128
B = 16
LL = 128
A = 64

NC = 2
NS = 16
NW = NC * NS
GSZ = 128
NGW = 80
EPAD = NW * NGW * GSZ
NG = EPAD // GSZ
NROWS = 10112
ZROWS = NROWS // NS
OROWS = 624
TAIL = N - NS * OROWS

_HIGH = jax.lax.Precision.HIGHEST


def _dot(a, b, dims):
    return lax.dot_general(a, b, (dims, ((), ())), precision=_HIGH,
                           preferred_element_type=jnp.float32)


NBUF = 2
IB = 40
NCHUNK = NGW // IB


def _sc_agg_body(x_hbm, src_hbm, dst_hbm, p0_hbm, p1_hbm,
                 agg, sbuf, dbuf, rows, sems):
    c = lax.axis_index("c")
    s = lax.axis_index("s")
    w = c * NS + s

    z16 = jnp.zeros((16,), jnp.float32)

    def zrow(r, carry):
        for col in range(0, HD, 16):
            rows[0][r, pl.ds(col, 16)] = z16
        return carry

    lax.fori_loop(0, GSZ, zrow, 0)
    nfull = ZROWS // GSZ
    for i in range(nfull):
        pltpu.sync_copy(rows[0], agg.at[pl.ds(s * ZROWS + i * GSZ, GSZ)])
    rem = ZROWS - nfull * GSZ
    if rem:
        pltpu.sync_copy(rows[0].at[pl.ds(0, rem)],
                        agg.at[pl.ds(s * ZROWS + nfull * GSZ, rem)])
    plsc.subcore_barrier()

    for ci in range(NCHUNK):
        cb = w * NGW + ci * IB
        pltpu.sync_copy(src_hbm.at[pl.ds(cb, IB)], sbuf)
        pltpu.sync_copy(dst_hbm.at[pl.ds(cb, IB)], dbuf)
        for b in range(NBUF):
            pltpu.make_async_copy(x_hbm.at[sbuf.at[b]], rows[b], sems[b]).start()

        def pair(j, carry):
            for b in range(NBUF):
                g = NBUF * j + b
                pltpu.make_async_copy(x_hbm.at[sbuf.at[g]], rows[b],
                                      sems[b]).wait()
                pltpu.sync_copy(rows[b], agg.at[dbuf.at[g]], add=True)

                @pl.when(g + NBUF < IB)
                def _(g=g, b=b):
                    pltpu.make_async_copy(x_hbm.at[sbuf.at[g + NBUF]], rows[b],
                                          sems[b]).start()
            return carry

        lax.fori_loop(0, IB // NBUF, pair, 0)

    plsc.subcore_barrier()

    @pl.when(c == 0)
    def _():
        pltpu.sync_copy(agg.at[pl.ds(s * OROWS, OROWS)],
                        p0_hbm.at[pl.ds(s * OROWS, OROWS)])

        @pl.when(s == 0)
        def _():
            pltpu.sync_copy(agg.at[pl.ds(NS * OROWS, TAIL)],
                            p0_hbm.at[pl.ds(NS * OROWS, TAIL)])

    @pl.when(c == 1)
    def _():
        pltpu.sync_copy(agg.at[pl.ds(s * OROWS, OROWS)],
                        p1_hbm.at[pl.ds(s * OROWS, OROWS)])

        @pl.when(s == 0)
        def _():
            pltpu.sync_copy(agg.at[pl.ds(NS * OROWS, TAIL)],
                            p1_hbm.at[pl.ds(NS * OROWS, TAIL)])


_sc_agg = pl.kernel(
    _sc_agg_body,
    out_type=(jax.ShapeDtypeStruct((N, HD), jnp.float32),
              jax.ShapeDtypeStruct((N, HD), jnp.float32)),
    mesh=plsc.VectorSubcoreMesh(core_axis_name="c", subcore_axis_name="s",
                                num_cores=NC, num_subcores=NS),
    scratch_types=[
        pltpu.VMEM_SHARED((NROWS, HD), jnp.float32),
        pltpu.VMEM((IB, GSZ), jnp.int32),
        pltpu.VMEM((IB, GSZ), jnp.int32),
        [pltpu.VMEM((GSZ, HD), jnp.float32)] * NBUF,
        [pltpu.SemaphoreType.DMA] * NBUF,
    ],
)


def _mlp_block(h, w1_ref, b1_ref, w2_ref, b2_ref):
    t = jnp.maximum(_dot(h, w1_ref[...], (((1,), (0,)))) + b1_ref[...], 0.0)
    return jnp.maximum(_dot(t, w2_ref[...], (((1,), (0,)))) + b2_ref[...], 0.0)


def _tc_mlp_body(x_ref, p0_ref, p1_ref, w1_ref, b1_ref, w2_ref, b2_ref, o_ref):
    h = x_ref[...] + p0_ref[...] + p1_ref[...]
    o_ref[...] = _mlp_block(h, w1_ref, b1_ref, w2_ref, b2_ref)


def _tc_head_body(h_ref, q0_ref, q1_ref, w1_ref, b1_ref, w2_ref, b2_ref,
                  batch_ref, hp_ref, cp_ref, wih_ref, whh_ref, bih_ref,
                  wf_ref, bf_ref, probs_ref, hn_ref, cn_ref,
                  pooled_acc, cnt_acc, BN):
    i = pl.program_id(0)

    @pl.when(i == 0)
    def _():
        pooled_acc[...] = jnp.zeros_like(pooled_acc)
        cnt_acc[...] = jnp.zeros_like(cnt_acc)

    h = h_ref[...] + q0_ref[...] + q1_ref[...]
    h2 = _mlp_block(h, w1_ref, b1_ref, w2_ref, b2_ref)

    mask = (batch_ref[0] == lax.broadcasted_iota(jnp.int32, (B, BN), 0))
    mask = mask.astype(jnp.float32)
    pooled_acc[...] += _dot(mask, h2, (((1,), (0,))))
    cnt_acc[...] += _dot(mask, jnp.ones((BN, HD), jnp.float32), (((1,), (0,))))

    @pl.when(i == pl.num_programs(0) - 1)
    def _():
        pooled = pooled_acc[...] / jnp.maximum(cnt_acc[...], 1.0)
        gates = (_dot(pooled, wih_ref[...], (((1,), (0,))))
                 + _dot(hp_ref[...], whh_ref[...], (((1,), (0,))))
                 + bih_ref[...])
        i_g = jax.nn.sigmoid(gates[:, 0 * LL:1 * LL])
        f_g = jax.nn.sigmoid(gates[:, 1 * LL:2 * LL])
        g_g = jnp.tanh(gates[:, 2 * LL:3 * LL])
        o_g = jax.nn.sigmoid(gates[:, 3 * LL:4 * LL])
        c_new = f_g * cp_ref[...] + i_g * g_g
        h_new = o_g * jnp.tanh(c_new)
        logits = _dot(h_new, wf_ref[...], (((1, ), (0,)))) + bf_ref[...]
        m = jnp.max(logits, axis=1, keepdims=True)
        e = jnp.exp(logits - m)
        probs_ref[...] = e / jnp.sum(e, axis=1, keepdims=True)
        hn_ref[...] = h_new
        cn_ref[...] = c_new


def _tc_mlp(x, p0, p1, w1, b1, w2, b2, nblk, bn):
    row = lambda i: (i, 0)
    full = lambda i: (0, 0)
    return pl.pallas_call(
        _tc_mlp_body,
        grid=(nblk,),
        in_specs=[pl.BlockSpec((bn, HD), row)] * 3 + [
            pl.BlockSpec((HD, HD), full), pl.BlockSpec((1, HD), full),
            pl.BlockSpec((HD, HD), full), pl.BlockSpec((1, HD), full),
        ],
        out_specs=pl.BlockSpec((bn, HD), row),
        out_shape=jax.ShapeDtypeStruct((N, HD), jnp.float32),
    )(x, p0, p1, w1, b1, w2, b2)


def _tc_head(h1, q0, q1, w1, b1, w2, b2, batch3d, hp, cp, wihT, whhT, bihs,
             wf, bf, nblk, bn):
    row = lambda i: (i, 0)
    full = lambda i: (0, 0)
    return pl.pallas_call(
        functools.partial(_tc_head_body, BN=bn),
        grid=(nblk,),
        in_specs=[pl.BlockSpec((bn, HD), row)] * 3 + [
            pl.BlockSpec((HD, HD), full), pl.BlockSpec((1, HD), full),
            pl.BlockSpec((HD, HD), full), pl.BlockSpec((1, HD), full),
            pl.BlockSpec((1, 1, bn), lambda i: (i, 0, 0)),
            pl.BlockSpec((B, LL), full), pl.BlockSpec((B, LL), full),
            pl.BlockSpec((HD, 4 * LL), full), pl.BlockSpec((LL, 4 * LL), full),
            pl.BlockSpec((1, 4 * LL), full),
            pl.BlockSpec((LL, A), full), pl.BlockSpec((1, A), full),
        ],
        out_specs=[pl.BlockSpec((B, A), full), pl.BlockSpec((B, LL), full),
                   pl.BlockSpec((B, LL), full)],
        out_shape=[jax.ShapeDtypeStruct((B, A), jnp.float32),
                   jax.ShapeDtypeStruct((B, LL), jnp.float32),
                   jax.ShapeDtypeStruct((B, LL), jnp.float32)],
        scratch_shapes=[pltpu.VMEM((B, HD), jnp.float32),
                        pltpu.VMEM((B, HD), jnp.float32)],
    )(h1, q0, q1, w1, b1, w2, b2, batch3d, hp, cp, wihT, whhT, bihs, wf, bf)


def kernel(x, edge_index, batch, h0, c0,
           W1a, b1a, W2a, b2a, W1b, b1b, W2b, b2b,
           W_ih, W_hh, b_ih, b_hh, Wf, bf):
    pad = EPAD - E
    padi = jnp.arange(pad, dtype=jnp.int32)
    src_p = jnp.concatenate([edge_index[0], (padi * 131) % N])
    dst_p = jnp.concatenate([edge_index[1], N + (padi % (NROWS - N))])
    src2d = src_p.reshape(NG, GSZ)
    dst2d = dst_p.reshape(NG, GSZ)

    nblk, bn = 10, 1000
    batch3d = batch.reshape(nblk, 1, bn)
    b1as, b2as = b1a.reshape(1, HD), b2a.reshape(1, HD)
    b1bs, b2bs = b1b.reshape(1, HD), b2b.reshape(1, HD)
    bihs = (b_ih + b_hh).reshape(1, 4 * LL)
    bfs = bf.reshape(1, A)

    p0, p1 = _sc_agg(x, src2d, dst2d)
    h1 = _tc_mlp(x, p0, p1, W1a, b1as, W2a, b2as, nblk, bn)

    q0, q1 = _sc_agg(h1, src2d, dst2d)
    probs, h_new, c_new = _tc_head(h1, q0, q1, W1b, b1bs, W2b, b2bs,
                                   batch3d, h0[0], c0[0], W_ih.T, W_hh.T,
                                   bihs, Wf, bfs, nblk, bn)
    return probs, h_new[None], c_new[None]


# speedup vs baseline: 11.8234x; 1.1722x over previous
import functools

import jax
import jax.numpy as jnp
from jax import lax
from jax.experimental import pallas as pl
from jax.experimental.pallas import tpu as pltpu
from jax.experimental.pallas import tpu_sc as plsc

N = 10000
E = 320000
HD = 128
B = 16
LL = 128
A = 64

NC = 2
NS = 16
NW = NC * NS
GSZ = 128
NGW = 80
EPAD = NW * NGW * GSZ
NG = EPAD // GSZ
NROWS = 10112
ZROWS = NROWS // NS
OROWS = 624
TAIL = N - NS * OROWS

_HIGH = jax.lax.Precision.DEFAULT


def _dot(a, b, dims):
    return lax.dot_general(a, b, (dims, ((), ())), precision=_HIGH,
                           preferred_element_type=jnp.float32)


NBUF = 2
IB = 40
NCHUNK = NGW // IB


def _sc_agg_body(x_hbm, src_hbm, dst_hbm, p0_hbm, p1_hbm,
                 agg, sbuf, dbuf, rows, sems):
    c = lax.axis_index("c")
    s = lax.axis_index("s")
    w = c * NS + s

    z16 = jnp.zeros((16,), jnp.float32)

    def zrow(r, carry):
        for col in range(0, HD, 16):
            rows[0][r, pl.ds(col, 16)] = z16
        return carry

    lax.fori_loop(0, GSZ, zrow, 0)
    nfull = ZROWS // GSZ
    for i in range(nfull):
        pltpu.sync_copy(rows[0], agg.at[pl.ds(s * ZROWS + i * GSZ, GSZ)])
    rem = ZROWS - nfull * GSZ
    if rem:
        pltpu.sync_copy(rows[0].at[pl.ds(0, rem)],
                        agg.at[pl.ds(s * ZROWS + nfull * GSZ, rem)])
    plsc.subcore_barrier()

    for ci in range(NCHUNK):
        cb = w * NGW + ci * IB
        pltpu.sync_copy(src_hbm.at[pl.ds(cb, IB)], sbuf)
        pltpu.sync_copy(dst_hbm.at[pl.ds(cb, IB)], dbuf)
        for b in range(NBUF):
            pltpu.make_async_copy(x_hbm.at[sbuf.at[b]], rows[b], sems[b]).start()

        def pair(j, carry):
            for b in range(NBUF):
                g = NBUF * j + b
                pltpu.make_async_copy(x_hbm.at[sbuf.at[g]], rows[b],
                                      sems[b]).wait()
                pltpu.sync_copy(rows[b], agg.at[dbuf.at[g]], add=True)

                @pl.when(g + NBUF < IB)
                def _(g=g, b=b):
                    pltpu.make_async_copy(x_hbm.at[sbuf.at[g + NBUF]], rows[b],
                                          sems[b]).start()
            return carry

        lax.fori_loop(0, IB // NBUF, pair, 0)

    plsc.subcore_barrier()

    @pl.when(c == 0)
    def _():
        pltpu.sync_copy(agg.at[pl.ds(s * OROWS, OROWS)],
                        p0_hbm.at[pl.ds(s * OROWS, OROWS)])

        @pl.when(s == 0)
        def _():
            pltpu.sync_copy(agg.at[pl.ds(NS * OROWS, TAIL)],
                            p0_hbm.at[pl.ds(NS * OROWS, TAIL)])

    @pl.when(c == 1)
    def _():
        pltpu.sync_copy(agg.at[pl.ds(s * OROWS, OROWS)],
                        p1_hbm.at[pl.ds(s * OROWS, OROWS)])

        @pl.when(s == 0)
        def _():
            pltpu.sync_copy(agg.at[pl.ds(NS * OROWS, TAIL)],
                            p1_hbm.at[pl.ds(NS * OROWS, TAIL)])


_sc_agg = pl.kernel(
    _sc_agg_body,
    out_type=(jax.ShapeDtypeStruct((N, HD), jnp.float32),
              jax.ShapeDtypeStruct((N, HD), jnp.float32)),
    mesh=plsc.VectorSubcoreMesh(core_axis_name="c", subcore_axis_name="s",
                                num_cores=NC, num_subcores=NS),
    scratch_types=[
        pltpu.VMEM_SHARED((NROWS, HD), jnp.float32),
        pltpu.VMEM((IB, GSZ), jnp.int32),
        pltpu.VMEM((IB, GSZ), jnp.int32),
        [pltpu.VMEM((GSZ, HD), jnp.float32)] * NBUF,
        [pltpu.SemaphoreType.DMA] * NBUF,
    ],
)


def _mlp_block(h, w1_ref, b1_ref, w2_ref, b2_ref):
    t = jnp.maximum(_dot(h, w1_ref[...], (((1,), (0,)))) + b1_ref[...], 0.0)
    return jnp.maximum(_dot(t, w2_ref[...], (((1,), (0,)))) + b2_ref[...], 0.0)


def _tc_mlp_body(x_ref, p0_ref, p1_ref, w1_ref, b1_ref, w2_ref, b2_ref, o_ref):
    h = x_ref[...] + p0_ref[...] + p1_ref[...]
    o_ref[...] = _mlp_block(h, w1_ref, b1_ref, w2_ref, b2_ref)


def _tc_head_body(h_ref, q0_ref, q1_ref, w1_ref, b1_ref, w2_ref, b2_ref,
                  batch_ref, hp_ref, cp_ref, wih_ref, whh_ref, bih_ref,
                  wf_ref, bf_ref, probs_ref, hn_ref, cn_ref,
                  pooled_acc, cnt_acc, BN):
    i = pl.program_id(0)

    @pl.when(i == 0)
    def _():
        pooled_acc[...] = jnp.zeros_like(pooled_acc)
        cnt_acc[...] = jnp.zeros_like(cnt_acc)

    h = h_ref[...] + q0_ref[...] + q1_ref[...]
    h2 = _mlp_block(h, w1_ref, b1_ref, w2_ref, b2_ref)

    mask = (batch_ref[0] == lax.broadcasted_iota(jnp.int32, (B, BN), 0))
    mask = mask.astype(jnp.float32)
    pooled_acc[...] += _dot(mask, h2, (((1,), (0,))))
    cnt_acc[...] += _dot(mask, jnp.ones((BN, HD), jnp.float32), (((1,), (0,))))

    @pl.when(i == pl.num_programs(0) - 1)
    def _():
        pooled = pooled_acc[...] / jnp.maximum(cnt_acc[...], 1.0)
        gates = (_dot(pooled, wih_ref[...], (((1,), (0,))))
                 + _dot(hp_ref[...], whh_ref[...], (((1,), (0,))))
                 + bih_ref[...])
        i_g = jax.nn.sigmoid(gates[:, 0 * LL:1 * LL])
        f_g = jax.nn.sigmoid(gates[:, 1 * LL:2 * LL])
        g_g = jnp.tanh(gates[:, 2 * LL:3 * LL])
        o_g = jax.nn.sigmoid(gates[:, 3 * LL:4 * LL])
        c_new = f_g * cp_ref[...] + i_g * g_g
        h_new = o_g * jnp.tanh(c_new)
        logits = _dot(h_new, wf_ref[...], (((1, ), (0,)))) + bf_ref[...]
        m = jnp.max(logits, axis=1, keepdims=True)
        e = jnp.exp(logits - m)
        probs_ref[...] = e / jnp.sum(e, axis=1, keepdims=True)
        hn_ref[...] = h_new
        cn_ref[...] = c_new


def _tc_mlp(x, p0, p1, w1, b1, w2, b2, nblk, bn):
    row = lambda i: (i, 0)
    full = lambda i: (0, 0)
    return pl.pallas_call(
        _tc_mlp_body,
        grid=(nblk,),
        in_specs=[pl.BlockSpec((bn, HD), row)] * 3 + [
            pl.BlockSpec((HD, HD), full), pl.BlockSpec((1, HD), full),
            pl.BlockSpec((HD, HD), full), pl.BlockSpec((1, HD), full),
        ],
        out_specs=pl.BlockSpec((bn, HD), row),
        out_shape=jax.ShapeDtypeStruct((N, HD), jnp.float32),
    )(x, p0, p1, w1, b1, w2, b2)


def _tc_head(h1, q0, q1, w1, b1, w2, b2, batch3d, hp, cp, wihT, whhT, bihs,
             wf, bf, nblk, bn):
    row = lambda i: (i, 0)
    full = lambda i: (0, 0)
    return pl.pallas_call(
        functools.partial(_tc_head_body, BN=bn),
        grid=(nblk,),
        in_specs=[pl.BlockSpec((bn, HD), row)] * 3 + [
            pl.BlockSpec((HD, HD), full), pl.BlockSpec((1, HD), full),
            pl.BlockSpec((HD, HD), full), pl.BlockSpec((1, HD), full),
            pl.BlockSpec((1, 1, bn), lambda i: (i, 0, 0)),
            pl.BlockSpec((B, LL), full), pl.BlockSpec((B, LL), full),
            pl.BlockSpec((HD, 4 * LL), full), pl.BlockSpec((LL, 4 * LL), full),
            pl.BlockSpec((1, 4 * LL), full),
            pl.BlockSpec((LL, A), full), pl.BlockSpec((1, A), full),
        ],
        out_specs=[pl.BlockSpec((B, A), full), pl.BlockSpec((B, LL), full),
                   pl.BlockSpec((B, LL), full)],
        out_shape=[jax.ShapeDtypeStruct((B, A), jnp.float32),
                   jax.ShapeDtypeStruct((B, LL), jnp.float32),
                   jax.ShapeDtypeStruct((B, LL), jnp.float32)],
        scratch_shapes=[pltpu.VMEM((B, HD), jnp.float32),
                        pltpu.VMEM((B, HD), jnp.float32)],
    )(h1, q0, q1, w1, b1, w2, b2, batch3d, hp, cp, wihT, whhT, bihs, wf, bf)


def kernel(x, edge_index, batch, h0, c0,
           W1a, b1a, W2a, b2a, W1b, b1b, W2b, b2b,
           W_ih, W_hh, b_ih, b_hh, Wf, bf):
    pad = EPAD - E
    padi = jnp.arange(pad, dtype=jnp.int32)
    src_p = jnp.concatenate([edge_index[0], (padi * 131) % N])
    dst_p = jnp.concatenate([edge_index[1], N + (padi % (NROWS - N))])
    src2d = src_p.reshape(NG, GSZ)
    dst2d = dst_p.reshape(NG, GSZ)

    nblk, bn = 10, 1000
    batch3d = batch.reshape(nblk, 1, bn)
    b1as, b2as = b1a.reshape(1, HD), b2a.reshape(1, HD)
    b1bs, b2bs = b1b.reshape(1, HD), b2b.reshape(1, HD)
    bihs = (b_ih + b_hh).reshape(1, 4 * LL)
    bfs = bf.reshape(1, A)

    p0, p1 = _sc_agg(x, src2d, dst2d)
    h1 = _tc_mlp(x, p0, p1, W1a, b1as, W2a, b2as, nblk, bn)

    q0, q1 = _sc_agg(h1, src2d, dst2d)
    probs, h_new, c_new = _tc_head(h1, q0, q1, W1b, b1bs, W2b, b2bs,
                                   batch3d, h0[0], c0[0], W_ih.T, W_hh.T,
                                   bihs, Wf, bfs, nblk, bn)
    return probs, h_new[None], c_new[None]


# speedup vs baseline: 11.9599x; 1.0115x over previous
import functools

import jax
import jax.numpy as jnp
from jax import lax
from jax.experimental import pallas as pl
from jax.experimental.pallas import tpu as pltpu
from jax.experimental.pallas import tpu_sc as plsc

N = 10000
E = 320000
HD = 128
B = 16
LL = 128
A = 64

NC = 2
NS = 16
NW = NC * NS
GSZ = 128
NGW = 81
EW = E // NW
PADW = NGW * GSZ - EW
NROWS = 10112
ZROWS = NROWS // NS
OROWS = 624
TAIL = N - NS * OROWS

_HIGH = jax.lax.Precision.DEFAULT


def _dot(a, b, dims):
    return lax.dot_general(a, b, (dims, ((), ())), precision=_HIGH,
                           preferred_element_type=jnp.float32)


NBUF = 3


def _sc_agg_body(x_hbm, src_hbm, dst_hbm, p0_hbm, p1_hbm,
                 agg, sidx, didx, rows, gsem, ssem, isem_s, isem_d):
    c = lax.axis_index("c")
    s = lax.axis_index("s")
    w = c * NS + s
    base = w * NGW

    z16 = jnp.zeros((16,), jnp.float32)

    def zrow(r, carry):
        for col in range(0, HD, 16):
            rows[0][r, pl.ds(col, 16)] = z16
        return carry

    lax.fori_loop(0, GSZ, zrow, 0)
    nfull = ZROWS // GSZ
    for i in range(nfull):
        pltpu.sync_copy(rows[0], agg.at[pl.ds(s * ZROWS + i * GSZ, GSZ)])
    rem = ZROWS - nfull * GSZ
    if rem:
        pltpu.sync_copy(rows[0].at[pl.ds(0, rem)],
                        agg.at[pl.ds(s * ZROWS + nfull * GSZ, rem)])

    def start_sidx(g, k):
        pltpu.make_async_copy(src_hbm.at[pl.ds((base + g) * GSZ, GSZ)],
                              sidx[k], isem_s[k]).start()

    def start_didx(g, k):
        pltpu.make_async_copy(dst_hbm.at[pl.ds((base + g) * GSZ, GSZ)],
                              didx[k], isem_d[k]).start()

    def wait_sidx(k):
        pltpu.make_async_copy(src_hbm.at[pl.ds(0, GSZ)], sidx[k],
                              isem_s[k]).wait()

    def wait_didx(k):
        pltpu.make_async_copy(dst_hbm.at[pl.ds(0, GSZ)], didx[k],
                              isem_d[k]).wait()

    def start_gather(k):
        pltpu.make_async_copy(x_hbm.at[sidx[k]], rows[k], gsem[k]).start()

    def wait_gather(k):
        pltpu.make_async_copy(x_hbm.at[pl.ds(0, GSZ)], rows[k], gsem[k]).wait()

    def do_scatter(k):
        pltpu.sync_copy(rows[k], agg.at[didx[k]], add=True)

    for k in range(NBUF):
        start_sidx(k, k)
    for k in range(NBUF - 1):
        start_didx(k, k)
    for k in range(NBUF - 1):
        wait_sidx(k)
        start_gather(k)

    plsc.subcore_barrier()

    def tri(j, carry):
        for k in range(NBUF):
            g = NBUF * j + k
            kn = (k + 2) % NBUF
            wait_gather(k)
            wait_didx(k)
            do_scatter(k)

            @pl.when(g + NBUF < NGW)
            def _(g=g, k=k):
                start_sidx(g + NBUF, k)

            @pl.when(g + 2 < NGW)
            def _(g=g, kn=kn):
                start_didx(g + 2, kn)
                wait_sidx(kn)
                start_gather(kn)
        return carry

    lax.fori_loop(0, NGW // NBUF, tri, 0)
    plsc.subcore_barrier()

    @pl.when(c == 0)
    def _():
        pltpu.sync_copy(agg.at[pl.ds(s * OROWS, OROWS)],
                        p0_hbm.at[pl.ds(s * OROWS, OROWS)])

        @pl.when(s == 0)
        def _():
            pltpu.sync_copy(agg.at[pl.ds(NS * OROWS, TAIL)],
                            p0_hbm.at[pl.ds(NS * OROWS, TAIL)])

    @pl.when(c == 1)
    def _():
        pltpu.sync_copy(agg.at[pl.ds(s * OROWS, OROWS)],
                        p1_hbm.at[pl.ds(s * OROWS, OROWS)])

        @pl.when(s == 0)
        def _():
            pltpu.sync_copy(agg.at[pl.ds(NS * OROWS, TAIL)],
                            p1_hbm.at[pl.ds(NS * OROWS, TAIL)])


_sc_agg = pl.kernel(
    _sc_agg_body,
    out_type=(jax.ShapeDtypeStruct((N, HD), jnp.float32),
              jax.ShapeDtypeStruct((N, HD), jnp.float32)),
    mesh=plsc.VectorSubcoreMesh(core_axis_name="c", subcore_axis_name="s",
                                num_cores=NC, num_subcores=NS),
    scratch_types=[
        pltpu.VMEM_SHARED((NROWS, HD), jnp.float32),
        [pltpu.VMEM((GSZ,), jnp.int32)] * NBUF,
        [pltpu.VMEM((GSZ,), jnp.int32)] * NBUF,
        [pltpu.VMEM((GSZ, HD), jnp.float32)] * NBUF,
        [pltpu.SemaphoreType.DMA] * NBUF,
        [pltpu.SemaphoreType.DMA] * NBUF,
        [pltpu.SemaphoreType.DMA] * NBUF,
        [pltpu.SemaphoreType.DMA] * NBUF,
    ],
)


def _mlp_block(h, w1_ref, b1_ref, w2_ref, b2_ref):
    t = jnp.maximum(_dot(h, w1_ref[...], (((1,), (0,)))) + b1_ref[...], 0.0)
    return jnp.maximum(_dot(t, w2_ref[...], (((1,), (0,)))) + b2_ref[...], 0.0)


def _tc_mlp_body(x_ref, p0_ref, p1_ref, w1_ref, b1_ref, w2_ref, b2_ref, o_ref):
    h = x_ref[...] + p0_ref[...] + p1_ref[...]
    o_ref[...] = _mlp_block(h, w1_ref, b1_ref, w2_ref, b2_ref)


def _tc_head_body(h_ref, q0_ref, q1_ref, w1_ref, b1_ref, w2_ref, b2_ref,
                  batch_ref, hp_ref, cp_ref, wih_ref, whh_ref, bih_ref,
                  wf_ref, bf_ref, probs_ref, hn_ref, cn_ref,
                  pooled_acc, cnt_acc, BN):
    i = pl.program_id(0)

    @pl.when(i == 0)
    def _():
        pooled_acc[...] = jnp.zeros_like(pooled_acc)
        cnt_acc[...] = jnp.zeros_like(cnt_acc)

    h = h_ref[...] + q0_ref[...] + q1_ref[...]
    h2 = _mlp_block(h, w1_ref, b1_ref, w2_ref, b2_ref)

    mask = (batch_ref[0] == lax.broadcasted_iota(jnp.int32, (B, BN), 0))
    mask = mask.astype(jnp.float32)
    pooled_acc[...] += _dot(mask, h2, (((1,), (0,))))
    cnt_acc[...] += _dot(mask, jnp.ones((BN, HD), jnp.float32), (((1,), (0,))))

    @pl.when(i == pl.num_programs(0) - 1)
    def _():
        pooled = pooled_acc[...] / jnp.maximum(cnt_acc[...], 1.0)
        gates = (_dot(pooled, wih_ref[...], (((1,), (0,))))
                 + _dot(hp_ref[...], whh_ref[...], (((1,), (0,))))
                 + bih_ref[...])
        i_g = jax.nn.sigmoid(gates[:, 0 * LL:1 * LL])
        f_g = jax.nn.sigmoid(gates[:, 1 * LL:2 * LL])
        g_g = jnp.tanh(gates[:, 2 * LL:3 * LL])
        o_g = jax.nn.sigmoid(gates[:, 3 * LL:4 * LL])
        c_new = f_g * cp_ref[...] + i_g * g_g
        h_new = o_g * jnp.tanh(c_new)
        logits = _dot(h_new, wf_ref[...], (((1, ), (0,)))) + bf_ref[...]
        m = jnp.max(logits, axis=1, keepdims=True)
        e = jnp.exp(logits - m)
        probs_ref[...] = e / jnp.sum(e, axis=1, keepdims=True)
        hn_ref[...] = h_new
        cn_ref[...] = c_new


def _tc_mlp(x, p0, p1, w1, b1, w2, b2, nblk, bn):
    row = lambda i: (i, 0)
    full = lambda i: (0, 0)
    return pl.pallas_call(
        _tc_mlp_body,
        grid=(nblk,),
        in_specs=[pl.BlockSpec((bn, HD), row)] * 3 + [
            pl.BlockSpec((HD, HD), full), pl.BlockSpec((1, HD), full),
            pl.BlockSpec((HD, HD), full), pl.BlockSpec((1, HD), full),
        ],
        out_specs=pl.BlockSpec((bn, HD), row),
        out_shape=jax.ShapeDtypeStruct((N, HD), jnp.float32),
    )(x, p0, p1, w1, b1, w2, b2)


def _tc_head(h1, q0, q1, w1, b1, w2, b2, batch3d, hp, cp, wihT, whhT, bihs,
             wf, bf, nblk, bn):
    row = lambda i: (i, 0)
    full = lambda i: (0, 0)
    return pl.pallas_call(
        functools.partial(_tc_head_body, BN=bn),
        grid=(nblk,),
        in_specs=[pl.BlockSpec((bn, HD), row)] * 3 + [
            pl.BlockSpec((HD, HD), full), pl.BlockSpec((1, HD), full),
            pl.BlockSpec((HD, HD), full), pl.BlockSpec((1, HD), full),
            pl.BlockSpec((1, 1, bn), lambda i: (i, 0, 0)),
            pl.BlockSpec((B, LL), full), pl.BlockSpec((B, LL), full),
            pl.BlockSpec((HD, 4 * LL), full), pl.BlockSpec((LL, 4 * LL), full),
            pl.BlockSpec((1, 4 * LL), full),
            pl.BlockSpec((LL, A), full), pl.BlockSpec((1, A), full),
        ],
        out_specs=[pl.BlockSpec((B, A), full), pl.BlockSpec((B, LL), full),
                   pl.BlockSpec((B, LL), full)],
        out_shape=[jax.ShapeDtypeStruct((B, A), jnp.float32),
                   jax.ShapeDtypeStruct((B, LL), jnp.float32),
                   jax.ShapeDtypeStruct((B, LL), jnp.float32)],
        scratch_shapes=[pltpu.VMEM((B, HD), jnp.float32),
                        pltpu.VMEM((B, HD), jnp.float32)],
    )(h1, q0, q1, w1, b1, w2, b2, batch3d, hp, cp, wihT, whhT, bihs, wf, bf)


def kernel(x, edge_index, batch, h0, c0,
           W1a, b1a, W2a, b2a, W1b, b1b, W2b, b2b,
           W_ih, W_hh, b_ih, b_hh, Wf, bf):
    padi = jnp.arange(NW * PADW, dtype=jnp.int32).reshape(NW, PADW)
    src_p = jnp.concatenate(
        [edge_index[0].reshape(NW, EW), (padi * 131) % N], axis=1).reshape(-1)
    dst_p = jnp.concatenate(
        [edge_index[1].reshape(NW, EW), N + (padi % (NROWS - N))],
        axis=1).reshape(-1)

    nblk, bn = 10, 1000
    batch3d = batch.reshape(nblk, 1, bn)
    b1as, b2as = b1a.reshape(1, HD), b2a.reshape(1, HD)
    b1bs, b2bs = b1b.reshape(1, HD), b2b.reshape(1, HD)
    bihs = (b_ih + b_hh).reshape(1, 4 * LL)
    bfs = bf.reshape(1, A)

    p0, p1 = _sc_agg(x, src_p, dst_p)
    h1 = _tc_mlp(x, p0, p1, W1a, b1as, W2a, b2as, nblk, bn)

    q0, q1 = _sc_agg(h1, src_p, dst_p)
    probs, h_new, c_new = _tc_head(h1, q0, q1, W1b, b1bs, W2b, b2bs,
                                   batch3d, h0[0], c0[0], W_ih.T, W_hh.T,
                                   bihs, Wf, bfs, nblk, bn)
    return probs, h_new[None], c_new[None]


# speedup vs baseline: 12.2289x; 1.0225x over previous
import functools

import jax
import jax.numpy as jnp
from jax import lax
from jax.experimental import pallas as pl
from jax.experimental.pallas import tpu as pltpu
from jax.experimental.pallas import tpu_sc as plsc

N = 10000
E = 320000
HD = 128
B = 16
LL = 128
A = 64

NC = 2
NS = 16
NW = NC * NS
GSZ = 128
EW = E // NW
NFG = EW // GSZ
TGS = EW - NFG * GSZ
NROWS = 10112
ZROWS = NROWS // NS
OROWS = 624
TAIL = N - NS * OROWS

_HIGH = jax.lax.Precision.DEFAULT


def _dot(a, b, dims):
    return lax.dot_general(a, b, (dims, ((), ())), precision=_HIGH,
                           preferred_element_type=jnp.float32)


NBUF = 3


def _sc_agg_body(x_hbm, src_hbm, dst_hbm, p0_hbm, p1_hbm,
                 agg, sidx, didx, tidx_s, tidx_d, rows,
                 gsem, ssem, isem_s, isem_d):
    c = lax.axis_index("c")
    s = lax.axis_index("s")
    w = c * NS + s
    base = w * EW

    z16 = jnp.zeros((16,), jnp.float32)

    def zrow(r, carry):
        for col in range(0, HD, 16):
            rows[0][r, pl.ds(col, 16)] = z16
        return carry

    lax.fori_loop(0, GSZ, zrow, 0)
    nfull = ZROWS // GSZ
    for i in range(nfull):
        pltpu.sync_copy(rows[0], agg.at[pl.ds(s * ZROWS + i * GSZ, GSZ)])
    rem = ZROWS - nfull * GSZ
    if rem:
        pltpu.sync_copy(rows[0].at[pl.ds(0, rem)],
                        agg.at[pl.ds(s * ZROWS + nfull * GSZ, rem)])

    def start_sidx(g, k):
        pltpu.make_async_copy(src_hbm.at[pl.ds(base + g * GSZ, GSZ)],
                              sidx[k], isem_s[k]).start()

    def start_didx(g, k):
        pltpu.make_async_copy(dst_hbm.at[pl.ds(base + g * GSZ, GSZ)],
                              didx[k], isem_d[k]).start()

    def wait_sidx(k):
        pltpu.make_async_copy(src_hbm.at[pl.ds(0, GSZ)], sidx[k],
                              isem_s[k]).wait()

    def wait_didx(k):
        pltpu.make_async_copy(dst_hbm.at[pl.ds(0, GSZ)], didx[k],
                              isem_d[k]).wait()

    def start_gather(k):
        pltpu.make_async_copy(x_hbm.at[sidx[k]], rows[k], gsem[k]).start()

    def wait_gather(k):
        pltpu.make_async_copy(x_hbm.at[pl.ds(0, GSZ)], rows[k], gsem[k]).wait()

    def do_scatter(k):
        pltpu.sync_copy(rows[k], agg.at[didx[k]], add=True)

    for k in range(NBUF):
        start_sidx(k, k)
    for k in range(NBUF - 1):
        start_didx(k, k)
    for k in range(NBUF - 1):
        wait_sidx(k)
        start_gather(k)

    plsc.subcore_barrier()

    def tri(j, carry):
        for k in range(NBUF):
            g = NBUF * j + k
            kn = (k + 2) % NBUF
            wait_gather(k)
            wait_didx(k)
            do_scatter(k)

            @pl.when(g + NBUF < NFG)
            def _(g=g, k=k):
                start_sidx(g + NBUF, k)

            @pl.when(g + 2 < NFG)
            def _(g=g, kn=kn):
                start_didx(g + 2, kn)
                wait_sidx(kn)
                start_gather(kn)
        return carry

    lax.fori_loop(0, NFG // NBUF, tri, 0)

    pltpu.sync_copy(src_hbm.at[pl.ds(base + NFG * GSZ, TGS)], tidx_s)
    pltpu.sync_copy(dst_hbm.at[pl.ds(base + NFG * GSZ, TGS)], tidx_d)
    pltpu.async_copy(x_hbm.at[tidx_s], rows[0].at[pl.ds(0, TGS)],
                     gsem[0]).wait()
    pltpu.sync_copy(rows[0].at[pl.ds(0, TGS)], agg.at[tidx_d], add=True)
    plsc.subcore_barrier()

    @pl.when(c == 0)
    def _():
        pltpu.sync_copy(agg.at[pl.ds(s * OROWS, OROWS)],
                        p0_hbm.at[pl.ds(s * OROWS, OROWS)])

        @pl.when(s == 0)
        def _():
            pltpu.sync_copy(agg.at[pl.ds(NS * OROWS, TAIL)],
                            p0_hbm.at[pl.ds(NS * OROWS, TAIL)])

    @pl.when(c == 1)
    def _():
        pltpu.sync_copy(agg.at[pl.ds(s * OROWS, OROWS)],
                        p1_hbm.at[pl.ds(s * OROWS, OROWS)])

        @pl.when(s == 0)
        def _():
            pltpu.sync_copy(agg.at[pl.ds(NS * OROWS, TAIL)],
                            p1_hbm.at[pl.ds(NS * OROWS, TAIL)])


_sc_agg = pl.kernel(
    _sc_agg_body,
    out_type=(jax.ShapeDtypeStruct((N, HD), jnp.float32),
              jax.ShapeDtypeStruct((N, HD), jnp.float32)),
    mesh=plsc.VectorSubcoreMesh(core_axis_name="c", subcore_axis_name="s",
                                num_cores=NC, num_subcores=NS),
    scratch_types=[
        pltpu.VMEM_SHARED((NROWS, HD), jnp.float32),
        [pltpu.VMEM((GSZ,), jnp.int32)] * NBUF,
        [pltpu.VMEM((GSZ,), jnp.int32)] * NBUF,
        pltpu.VMEM((TGS,), jnp.int32),
        pltpu.VMEM((TGS,), jnp.int32),
        [pltpu.VMEM((GSZ, HD), jnp.float32)] * NBUF,
        [pltpu.SemaphoreType.DMA] * NBUF,
        [pltpu.SemaphoreType.DMA] * NBUF,
        [pltpu.SemaphoreType.DMA] * NBUF,
        [pltpu.SemaphoreType.DMA] * NBUF,
    ],
)


def _mlp_block(h, w1_ref, b1_ref, w2_ref, b2_ref):
    t = jnp.maximum(_dot(h, w1_ref[...], (((1,), (0,)))) + b1_ref[...], 0.0)
    return jnp.maximum(_dot(t, w2_ref[...], (((1,), (0,)))) + b2_ref[...], 0.0)


def _tc_mlp_body(x_ref, p0_ref, p1_ref, w1_ref, b1_ref, w2_ref, b2_ref, o_ref):
    h = x_ref[...] + p0_ref[...] + p1_ref[...]
    o_ref[...] = _mlp_block(h, w1_ref, b1_ref, w2_ref, b2_ref)


def _tc_head_body(h_ref, q0_ref, q1_ref, w1_ref, b1_ref, w2_ref, b2_ref,
                  batch_ref, hp_ref, cp_ref, wih_ref, whh_ref, bih_ref,
                  wf_ref, bf_ref, probs_ref, hn_ref, cn_ref,
                  pooled_acc, cnt_acc, BN):
    i = pl.program_id(0)

    @pl.when(i == 0)
    def _():
        pooled_acc[...] = jnp.zeros_like(pooled_acc)
        cnt_acc[...] = jnp.zeros_like(cnt_acc)

    h = h_ref[...] + q0_ref[...] + q1_ref[...]
    h2 = _mlp_block(h, w1_ref, b1_ref, w2_ref, b2_ref)

    mask = (batch_ref[0] == lax.broadcasted_iota(jnp.int32, (B, BN), 0))
    mask = mask.astype(jnp.float32)
    pooled_acc[...] += _dot(mask, h2, (((1,), (0,))))
    cnt_acc[...] += _dot(mask, jnp.ones((BN, HD), jnp.float32), (((1,), (0,))))

    @pl.when(i == pl.num_programs(0) - 1)
    def _():
        pooled = pooled_acc[...] / jnp.maximum(cnt_acc[...], 1.0)
        gates = (_dot(pooled, wih_ref[...], (((1,), (0,))))
                 + _dot(hp_ref[...], whh_ref[...], (((1,), (0,))))
                 + bih_ref[...])
        i_g = jax.nn.sigmoid(gates[:, 0 * LL:1 * LL])
        f_g = jax.nn.sigmoid(gates[:, 1 * LL:2 * LL])
        g_g = jnp.tanh(gates[:, 2 * LL:3 * LL])
        o_g = jax.nn.sigmoid(gates[:, 3 * LL:4 * LL])
        c_new = f_g * cp_ref[...] + i_g * g_g
        h_new = o_g * jnp.tanh(c_new)
        logits = _dot(h_new, wf_ref[...], (((1, ), (0,)))) + bf_ref[...]
        m = jnp.max(logits, axis=1, keepdims=True)
        e = jnp.exp(logits - m)
        probs_ref[...] = e / jnp.sum(e, axis=1, keepdims=True)
        hn_ref[...] = h_new
        cn_ref[...] = c_new


def _tc_mlp(x, p0, p1, w1, b1, w2, b2, nblk, bn):
    row = lambda i: (i, 0)
    full = lambda i: (0, 0)
    return pl.pallas_call(
        _tc_mlp_body,
        grid=(nblk,),
        in_specs=[pl.BlockSpec((bn, HD), row)] * 3 + [
            pl.BlockSpec((HD, HD), full), pl.BlockSpec((1, HD), full),
            pl.BlockSpec((HD, HD), full), pl.BlockSpec((1, HD), full),
        ],
        out_specs=pl.BlockSpec((bn, HD), row),
        out_shape=jax.ShapeDtypeStruct((N, HD), jnp.float32),
    )(x, p0, p1, w1, b1, w2, b2)


def _tc_head(h1, q0, q1, w1, b1, w2, b2, batch3d, hp, cp, wihT, whhT, bihs,
             wf, bf, nblk, bn):
    row = lambda i: (i, 0)
    full = lambda i: (0, 0)
    return pl.pallas_call(
        functools.partial(_tc_head_body, BN=bn),
        grid=(nblk,),
        in_specs=[pl.BlockSpec((bn, HD), row)] * 3 + [
            pl.BlockSpec((HD, HD), full), pl.BlockSpec((1, HD), full),
            pl.BlockSpec((HD, HD), full), pl.BlockSpec((1, HD), full),
            pl.BlockSpec((1, 1, bn), lambda i: (i, 0, 0)),
            pl.BlockSpec((B, LL), full), pl.BlockSpec((B, LL), full),
            pl.BlockSpec((HD, 4 * LL), full), pl.BlockSpec((LL, 4 * LL), full),
            pl.BlockSpec((1, 4 * LL), full),
            pl.BlockSpec((LL, A), full), pl.BlockSpec((1, A), full),
        ],
        out_specs=[pl.BlockSpec((B, A), full), pl.BlockSpec((B, LL), full),
                   pl.BlockSpec((B, LL), full)],
        out_shape=[jax.ShapeDtypeStruct((B, A), jnp.float32),
                   jax.ShapeDtypeStruct((B, LL), jnp.float32),
                   jax.ShapeDtypeStruct((B, LL), jnp.float32)],
        scratch_shapes=[pltpu.VMEM((B, HD), jnp.float32),
                        pltpu.VMEM((B, HD), jnp.float32)],
    )(h1, q0, q1, w1, b1, w2, b2, batch3d, hp, cp, wihT, whhT, bihs, wf, bf)


def kernel(x, edge_index, batch, h0, c0,
           W1a, b1a, W2a, b2a, W1b, b1b, W2b, b2b,
           W_ih, W_hh, b_ih, b_hh, Wf, bf):
    src_p = edge_index[0]
    dst_p = edge_index[1]

    nblk, bn = 10, 1000
    batch3d = batch.reshape(nblk, 1, bn)
    b1as, b2as = b1a.reshape(1, HD), b2a.reshape(1, HD)
    b1bs, b2bs = b1b.reshape(1, HD), b2b.reshape(1, HD)
    bihs = (b_ih + b_hh).reshape(1, 4 * LL)
    bfs = bf.reshape(1, A)

    p0, p1 = _sc_agg(x, src_p, dst_p)
    h1 = _tc_mlp(x, p0, p1, W1a, b1as, W2a, b2as, nblk, bn)

    q0, q1 = _sc_agg(h1, src_p, dst_p)
    probs, h_new, c_new = _tc_head(h1, q0, q1, W1b, b1bs, W2b, b2bs,
                                   batch3d, h0[0], c0[0], W_ih.T, W_hh.T,
                                   bihs, Wf, bfs, nblk, bn)
    return probs, h_new[None], c_new[None]


# speedup vs baseline: 12.5204x; 1.0238x over previous
import functools

import jax
import jax.numpy as jnp
from jax import lax
from jax.experimental import pallas as pl
from jax.experimental.pallas import tpu as pltpu
from jax.experimental.pallas import tpu_sc as plsc

N = 10000
E = 320000
HD = 128
B = 16
LL = 128
A = 64

NC = 2
NS = 16
NW = NC * NS
GSZ = 128
EW = E // NW
NFG = EW // GSZ
TGS = EW - NFG * GSZ
NROWS = 10112
ZROWS = NROWS // NS
OROWS = 624
TAIL = N - NS * OROWS

_HIGH = jax.lax.Precision.DEFAULT


def _dot(a, b, dims):
    return lax.dot_general(a, b, (dims, ((), ())), precision=_HIGH,
                           preferred_element_type=jnp.float32)


NBUF = 3


def _sc_agg_body(x_hbm, src_hbm, dst_hbm, p0_hbm, p1_hbm,
                 agg, sidx, didx, tidx_s, tidx_d, rows,
                 gsem, ssem, isem_s, isem_d):
    c = lax.axis_index("c")
    s = lax.axis_index("s")
    w = c * NS + s
    base = w * EW

    z16 = jnp.zeros((16,), jnp.float32)

    def zrow(r, carry):
        for col in range(0, HD, 16):
            rows[0][r, pl.ds(col, 16)] = z16
        return carry

    lax.fori_loop(0, GSZ, zrow, 0)
    nfull = ZROWS // GSZ
    for i in range(nfull):
        pltpu.sync_copy(rows[0], agg.at[pl.ds(s * ZROWS + i * GSZ, GSZ)])
    rem = ZROWS - nfull * GSZ
    if rem:
        pltpu.sync_copy(rows[0].at[pl.ds(0, rem)],
                        agg.at[pl.ds(s * ZROWS + nfull * GSZ, rem)])

    def start_sidx(g, k):
        pltpu.make_async_copy(src_hbm.at[pl.ds(base + g * GSZ, GSZ)],
                              sidx[k], isem_s[k]).start()

    def start_didx(g, k):
        pltpu.make_async_copy(dst_hbm.at[pl.ds(base + g * GSZ, GSZ)],
                              didx[k], isem_d[k]).start()

    def wait_sidx(k):
        pltpu.make_async_copy(src_hbm.at[pl.ds(0, GSZ)], sidx[k],
                              isem_s[k]).wait()

    def wait_didx(k):
        pltpu.make_async_copy(dst_hbm.at[pl.ds(0, GSZ)], didx[k],
                              isem_d[k]).wait()

    def start_gather(k):
        pltpu.make_async_copy(x_hbm.at[sidx[k]], rows[k], gsem[k]).start()

    def wait_gather(k):
        pltpu.make_async_copy(x_hbm.at[pl.ds(0, GSZ)], rows[k], gsem[k]).wait()

    def do_scatter(k):
        pltpu.sync_copy(rows[k], agg.at[didx[k]], add=True)

    for k in range(NBUF):
        start_sidx(k, k)
    for k in range(NBUF - 1):
        start_didx(k, k)
    for k in range(NBUF - 1):
        wait_sidx(k)
        start_gather(k)

    plsc.subcore_barrier()

    def tri(j, carry):
        for k in range(NBUF):
            g = NBUF * j + k
            kn = (k + 2) % NBUF
            wait_gather(k)
            wait_didx(k)
            do_scatter(k)

            @pl.when(g + NBUF < NFG)
            def _(g=g, k=k):
                start_sidx(g + NBUF, k)

            @pl.when(g + 2 < NFG)
            def _(g=g, kn=kn):
                start_didx(g + 2, kn)
                wait_sidx(kn)
                start_gather(kn)
        return carry

    lax.fori_loop(0, NFG // NBUF, tri, 0)

    pltpu.sync_copy(src_hbm.at[pl.ds(base + NFG * GSZ, TGS)], tidx_s)
    pltpu.sync_copy(dst_hbm.at[pl.ds(base + NFG * GSZ, TGS)], tidx_d)
    pltpu.async_copy(x_hbm.at[tidx_s], rows[0].at[pl.ds(0, TGS)],
                     gsem[0]).wait()
    pltpu.sync_copy(rows[0].at[pl.ds(0, TGS)], agg.at[tidx_d], add=True)
    plsc.subcore_barrier()

    @pl.when(c == 0)
    def _():
        pltpu.sync_copy(agg.at[pl.ds(s * OROWS, OROWS)],
                        p0_hbm.at[pl.ds(s * OROWS, OROWS)])

        @pl.when(s == 0)
        def _():
            pltpu.sync_copy(agg.at[pl.ds(NS * OROWS, TAIL)],
                            p0_hbm.at[pl.ds(NS * OROWS, TAIL)])

    @pl.when(c == 1)
    def _():
        pltpu.sync_copy(agg.at[pl.ds(s * OROWS, OROWS)],
                        p1_hbm.at[pl.ds(s * OROWS, OROWS)])

        @pl.when(s == 0)
        def _():
            pltpu.sync_copy(agg.at[pl.ds(NS * OROWS, TAIL)],
                            p1_hbm.at[pl.ds(NS * OROWS, TAIL)])


_sc_agg = pl.kernel(
    _sc_agg_body,
    out_type=(jax.ShapeDtypeStruct((N, HD), jnp.float32),
              jax.ShapeDtypeStruct((N, HD), jnp.float32)),
    mesh=plsc.VectorSubcoreMesh(core_axis_name="c", subcore_axis_name="s",
                                num_cores=NC, num_subcores=NS),
    scratch_types=[
        pltpu.VMEM_SHARED((NROWS, HD), jnp.float32),
        [pltpu.VMEM((GSZ,), jnp.int32)] * NBUF,
        [pltpu.VMEM((GSZ,), jnp.int32)] * NBUF,
        pltpu.VMEM((TGS,), jnp.int32),
        pltpu.VMEM((TGS,), jnp.int32),
        [pltpu.VMEM((GSZ, HD), jnp.float32)] * NBUF,
        [pltpu.SemaphoreType.DMA] * NBUF,
        [pltpu.SemaphoreType.DMA] * NBUF,
        [pltpu.SemaphoreType.DMA] * NBUF,
        [pltpu.SemaphoreType.DMA] * NBUF,
    ],
)


def _mlp_block(h, w1_ref, b1_ref, w2_ref, b2_ref):
    t = jnp.maximum(_dot(h, w1_ref[...], (((1,), (0,)))) + b1_ref[...], 0.0)
    return jnp.maximum(_dot(t, w2_ref[...], (((1,), (0,)))) + b2_ref[...], 0.0)


def _tc_mlp_body(x_ref, p0_ref, p1_ref, w1_ref, b1_ref, w2_ref, b2_ref, o_ref):
    h = x_ref[...] + p0_ref[...] + p1_ref[...]
    o_ref[...] = _mlp_block(h, w1_ref, b1_ref, w2_ref, b2_ref)


def _tc_head_body(h_ref, q0_ref, q1_ref, w1_ref, b1_ref, w2_ref, b2_ref,
                  batch_ref, hp_ref, cp_ref, wih_ref, whh_ref, bih_ref,
                  wf_ref, bf_ref, probs_ref, hn_ref, cn_ref,
                  pooled_acc, cnt_acc, BN):
    i = pl.program_id(0)

    @pl.when(i == 0)
    def _():
        pooled_acc[...] = jnp.zeros_like(pooled_acc)
        cnt_acc[...] = jnp.zeros_like(cnt_acc)

    h = h_ref[...] + q0_ref[...] + q1_ref[...]
    h2 = _mlp_block(h, w1_ref, b1_ref, w2_ref, b2_ref)

    mask = (batch_ref[0] == lax.broadcasted_iota(jnp.int32, (B, BN), 0))
    mask = mask.astype(jnp.float32)
    pooled_acc[...] += _dot(mask, h2, (((1,), (0,))))
    cnt_acc[...] += _dot(mask, jnp.ones((BN, HD), jnp.float32), (((1,), (0,))))

    @pl.when(i == pl.num_programs(0) - 1)
    def _():
        pooled = pooled_acc[...] / jnp.maximum(cnt_acc[...], 1.0)
        gates = (_dot(pooled, wih_ref[...], (((1,), (0,))))
                 + _dot(hp_ref[...], whh_ref[...], (((1,), (0,))))
                 + bih_ref[...])
        i_g = jax.nn.sigmoid(gates[:, 0 * LL:1 * LL])
        f_g = jax.nn.sigmoid(gates[:, 1 * LL:2 * LL])
        g_g = jnp.tanh(gates[:, 2 * LL:3 * LL])
        o_g = jax.nn.sigmoid(gates[:, 3 * LL:4 * LL])
        c_new = f_g * cp_ref[...] + i_g * g_g
        h_new = o_g * jnp.tanh(c_new)
        logits = _dot(h_new, wf_ref[...], (((1, ), (0,)))) + bf_ref[...]
        m = jnp.max(logits, axis=1, keepdims=True)
        e = jnp.exp(logits - m)
        probs_ref[...] = e / jnp.sum(e, axis=1, keepdims=True)
        hn_ref[...] = h_new
        cn_ref[...] = c_new


def _tc_mlp(x, p0, p1, w1, b1, w2, b2, nblk, bn):
    row = lambda i: (i, 0)
    full = lambda i: (0, 0)
    return pl.pallas_call(
        _tc_mlp_body,
        grid=(nblk,),
        in_specs=[pl.BlockSpec((bn, HD), row)] * 3 + [
            pl.BlockSpec((HD, HD), full), pl.BlockSpec((1, HD), full),
            pl.BlockSpec((HD, HD), full), pl.BlockSpec((1, HD), full),
        ],
        out_specs=pl.BlockSpec((bn, HD), row),
        out_shape=jax.ShapeDtypeStruct((N, HD), jnp.float32),
    )(x, p0, p1, w1, b1, w2, b2)


def _tc_head(h1, q0, q1, w1, b1, w2, b2, batch3d, hp, cp, wihT, whhT, bihs,
             wf, bf, nblk, bn):
    row = lambda i: (i, 0)
    full = lambda i: (0, 0)
    return pl.pallas_call(
        functools.partial(_tc_head_body, BN=bn),
        grid=(nblk,),
        in_specs=[pl.BlockSpec((bn, HD), row)] * 3 + [
            pl.BlockSpec((HD, HD), full), pl.BlockSpec((1, HD), full),
            pl.BlockSpec((HD, HD), full), pl.BlockSpec((1, HD), full),
            pl.BlockSpec((1, 1, bn), lambda i: (i, 0, 0)),
            pl.BlockSpec((B, LL), full), pl.BlockSpec((B, LL), full),
            pl.BlockSpec((HD, 4 * LL), full), pl.BlockSpec((LL, 4 * LL), full),
            pl.BlockSpec((1, 4 * LL), full),
            pl.BlockSpec((LL, A), full), pl.BlockSpec((1, A), full),
        ],
        out_specs=[pl.BlockSpec((B, A), full), pl.BlockSpec((B, LL), full),
                   pl.BlockSpec((B, LL), full)],
        out_shape=[jax.ShapeDtypeStruct((B, A), jnp.float32),
                   jax.ShapeDtypeStruct((B, LL), jnp.float32),
                   jax.ShapeDtypeStruct((B, LL), jnp.float32)],
        scratch_shapes=[pltpu.VMEM((B, HD), jnp.float32),
                        pltpu.VMEM((B, HD), jnp.float32)],
    )(h1, q0, q1, w1, b1, w2, b2, batch3d, hp, cp, wihT, whhT, bihs, wf, bf)


def kernel(x, edge_index, batch, h0, c0,
           W1a, b1a, W2a, b2a, W1b, b1b, W2b, b2b,
           W_ih, W_hh, b_ih, b_hh, Wf, bf):
    src_p = edge_index[0]
    dst_p = edge_index[1]

    nblk, bn = 5, 2000
    batch3d = batch.reshape(nblk, 1, bn)
    b1as, b2as = b1a.reshape(1, HD), b2a.reshape(1, HD)
    b1bs, b2bs = b1b.reshape(1, HD), b2b.reshape(1, HD)
    bihs = (b_ih + b_hh).reshape(1, 4 * LL)
    bfs = bf.reshape(1, A)

    p0, p1 = _sc_agg(x, src_p, dst_p)
    h1 = _tc_mlp(x, p0, p1, W1a, b1as, W2a, b2as, nblk, bn)

    q0, q1 = _sc_agg(h1, src_p, dst_p)
    probs, h_new, c_new = _tc_head(h1, q0, q1, W1b, b1bs, W2b, b2bs,
                                   batch3d, h0[0], c0[0], W_ih.T, W_hh.T,
                                   bihs, Wf, bfs, nblk, bn)
    return probs, h_new[None], c_new[None]
